# baseline (speedup 1.0000x reference)
import functools

import jax
import jax.numpy as jnp
from jax import lax
from jax.experimental import pallas as pl
from jax.experimental.pallas import tpu as pltpu
from jax.experimental.pallas import tpu_sc as plsc

_N = 10000
_D = 128
_E = 320000

_NC = 2
_NS = 16
_HALF = _D // 2

_CHUNK = 256
_IDX_ROWS = _CHUNK // 128
_EPT = 20480
_E_PAD = _EPT * _NS
_STEPS = _EPT // _CHUNK
_NPT = _N // _NS
_ACC_ROWS = _N + 16

_HIST_ROWS = _N // 16
_HIST_PAD = _HIST_ROWS + 8

_BN = 2000


def _deg_body(c0_2d, c1_2d, h0_out, h1_out, colbuf, hist):
    c = lax.axis_index("c")
    s = lax.axis_index("s")
    ones = jnp.full((16,), 1.0, dtype=jnp.float32)

    def run(col2d, hout):
        def zero_row(i, carry):
            hist[i] = jnp.zeros((16,), dtype=jnp.float32)
            return carry
        lax.fori_loop(0, _HIST_PAD, zero_row, 0)

        def step(t, carry):
            base = s * (_EPT // 128) + t * 16
            pltpu.sync_copy(col2d.at[pl.ds(base, 16)], colbuf)
            for j in range(16):
                for k in range(8):
                    v = colbuf[j, pl.ds(k * 16, 16)]
                    r = lax.shift_right_logical(v, 4)
                    cc = lax.bitwise_and(v, 15)
                    plsc.addupdate_scatter(hist, [r, cc], ones)
            return carry
        lax.fori_loop(0, _EPT // (16 * 128), step, 0)
        pltpu.sync_copy(hist.at[pl.ds(0, _HIST_ROWS)], hout.at[s])

    @pl.when(c == 0)
    def _():
        run(c0_2d, h0_out)

    @pl.when(c == 1)
    def _():
        run(c1_2d, h1_out)


def _deg_call(c0_2d, c1_2d):
    mesh = plsc.VectorSubcoreMesh(core_axis_name="c", subcore_axis_name="s", num_cores=_NC, num_subcores=_NS)
    f = pl.kernel(
        _deg_body,
        out_type=[
            jax.ShapeDtypeStruct((_NS, _HIST_ROWS, 16), jnp.float32),
            jax.ShapeDtypeStruct((_NS, _HIST_ROWS, 16), jnp.float32),
        ],
        mesh=mesh,
        scratch_types=[
            pltpu.VMEM((16, 128), jnp.int32),
            pltpu.VMEM((_HIST_PAD, 16), jnp.float32),
        ],
        compiler_params=pltpu.CompilerParams(needs_layout_passes=False),
    )
    return f(c0_2d, c1_2d)


def _prop_body(y0lo, y0hi, y1lo, y1hi, r0_2d, c0_2d, r1_2d, c1_2d,
               o0lo, o0hi, o1lo, o1hi,
               colbuf, rowbuf, rowsbuf, ysp, acc, gsem, ssem):
    c = lax.axis_index("c")
    s = lax.axis_index("s")

    def run_rel(y_hbm, out_hbm, acc, r2d, c2d):
        pltpu.sync_copy(y_hbm.at[pl.ds(s * _NPT, _NPT)],
                        ysp.at[pl.ds(s * _NPT, _NPT)])
        pltpu.sync_copy(y_hbm.at[pl.ds(s * _NPT, _NPT)],
                        acc.at[pl.ds(s * _NPT, _NPT)])
        plsc.subcore_barrier()

        def step(t, carry):
            p = lax.rem(t, 2)
            ioff = p * _IDX_ROWS
            roff = p * _CHUNK
            base = s * (_EPT // 128) + t * _IDX_ROWS
            pltpu.sync_copy(c2d.at[pl.ds(base, _IDX_ROWS)],
                            colbuf.at[pl.ds(ioff, _IDX_ROWS)])
            pltpu.sync_copy(r2d.at[pl.ds(base, _IDX_ROWS)],
                            rowbuf.at[pl.ds(ioff, _IDX_ROWS)])
            gh = [
                pltpu.async_copy(ysp.at[colbuf.at[ioff + j]],
                                 rowsbuf.at[pl.ds(roff + j * 128, 128)], gsem)
                for j in range(_IDX_ROWS)
            ]

            @pl.when(t > 0)
            def _():
                q = 1 - p
                qi = q * _IDX_ROWS
                qr = q * _CHUNK
                sh = [
                    pltpu.async_copy(rowsbuf.at[pl.ds(qr + j * 128, 128)],
                                     acc.at[rowbuf.at[qi + j]], ssem, add=True)
                    for j in range(_IDX_ROWS)
                ]
                for h in sh:
                    h.wait()

            for h in gh:
                h.wait()
            return carry

        lax.fori_loop(0, _STEPS, step, 0)

        lp = (_STEPS - 1) % 2
        sh = [
            pltpu.async_copy(rowsbuf.at[pl.ds(lp * _CHUNK + j * 128, 128)],
                             acc.at[rowbuf.at[lp * _IDX_ROWS + j]], ssem,
                             add=True)
            for j in range(_IDX_ROWS)
        ]
        for h in sh:
            h.wait()
        plsc.subcore_barrier()
        pltpu.sync_copy(acc.at[pl.ds(s * _NPT, _NPT)],
                        out_hbm.at[pl.ds(s * _NPT, _NPT)])

    @pl.when(c == 0)
    def _():
        run_rel(y0lo, o0lo, acc, r0_2d, c0_2d)
        run_rel(y1lo, o1lo, acc, r1_2d, c1_2d)

    @pl.when(c == 1)
    def _():
        run_rel(y0hi, o0hi, acc, r0_2d, c0_2d)
        run_rel(y1hi, o1hi, acc, r1_2d, c1_2d)


def _prop_call(y0lo, y0hi, y1lo, y1hi, r0_2d, c0_2d, r1_2d, c1_2d):
    mesh = plsc.VectorSubcoreMesh(core_axis_name="c", subcore_axis_name="s", num_cores=_NC, num_subcores=_NS)
    half = jax.ShapeDtypeStruct((_N, _HALF), jnp.float32)
    f = pl.kernel(
        _prop_body,
        out_type=[half, half, half, half],
        mesh=mesh,
        scratch_types=[
            pltpu.VMEM((2 * _IDX_ROWS, 128), jnp.int32),
            pltpu.VMEM((2 * _IDX_ROWS, 128), jnp.int32),
            pltpu.VMEM((2 * _CHUNK, _HALF), jnp.float32),
            pltpu.VMEM_SHARED((_N, _HALF), jnp.float32),
            pltpu.VMEM_SHARED((_ACC_ROWS, _HALF), jnp.float32),
            pltpu.SemaphoreType.DMA,
            pltpu.SemaphoreType.DMA,
        ],
        compiler_params=pltpu.CompilerParams(use_tc_tiling_on_sc=False),
    )
    return f(y0lo, y0hi, y1lo, y1hi, r0_2d, c0_2d, r1_2d, c1_2d)


def _prep_body(h0_ref, h1_ref, x_ref,
               y0lo, y0hi, y1lo, y1hi, d0_ref, d1_ref):
    deg0 = 1.0 + jnp.sum(h0_ref[...], axis=1, keepdims=True)
    deg1 = 1.0 + jnp.sum(h1_ref[...], axis=1, keepdims=True)
    dinv0 = jnp.where(deg0 > 0, lax.rsqrt(deg0), 0.0)
    dinv1 = jnp.where(deg1 > 0, lax.rsqrt(deg1), 0.0)
    x = x_ref[...]
    y0 = x * dinv0
    y1 = x * dinv1
    y0lo[...] = y0[:, :_HALF]
    y0hi[...] = y0[:, _HALF:]
    y1lo[...] = y1[:, :_HALF]
    y1hi[...] = y1[:, _HALF:]
    d0_ref[...] = dinv0
    d1_ref[...] = dinv1


def _prep_call(hp0, hp1, x):
    grid = (_N // _BN,)
    half_spec = pl.BlockSpec((_BN, _HALF), lambda i: (i, 0))
    f = pl.pallas_call(
        _prep_body,
        grid=grid,
        in_specs=[
            pl.BlockSpec((_BN, 16), lambda i: (i, 0)),
            pl.BlockSpec((_BN, 16), lambda i: (i, 0)),
            pl.BlockSpec((_BN, _D), lambda i: (i, 0)),
        ],
        out_specs=[
            half_spec, half_spec, half_spec, half_spec,
            pl.BlockSpec((_BN, 1), lambda i: (i, 0)),
            pl.BlockSpec((_BN, 1), lambda i: (i, 0)),
        ],
        out_shape=[
            jax.ShapeDtypeStruct((_N, _HALF), jnp.float32),
            jax.ShapeDtypeStruct((_N, _HALF), jnp.float32),
            jax.ShapeDtypeStruct((_N, _HALF), jnp.float32),
            jax.ShapeDtypeStruct((_N, _HALF), jnp.float32),
            jax.ShapeDtypeStruct((_N, 1), jnp.float32),
            jax.ShapeDtypeStruct((_N, 1), jnp.float32),
        ],
    )
    return f(hp0, hp1, x)


def _mid_body(o0lo, o0hi, o1lo, o1hi, d0_ref, d1_ref,
              z0lo, z0hi, z1lo, z1hi):
    s0 = d0_ref[...] * d0_ref[...]
    s1 = d1_ref[...] * d1_ref[...]
    z0lo[...] = o0lo[...] * s0
    z0hi[...] = o0hi[...] * s0
    z1lo[...] = o1lo[...] * s1
    z1hi[...] = o1hi[...] * s1


def _mid_call(o0lo, o0hi, o1lo, o1hi, d0, d1):
    grid = (_N // _BN,)
    half_spec = pl.BlockSpec((_BN, _HALF), lambda i: (i, 0))
    dspec = pl.BlockSpec((_BN, 1), lambda i: (i, 0))
    half = jax.ShapeDtypeStruct((_N, _HALF), jnp.float32)
    f = pl.pallas_call(
        _mid_body,
        grid=grid,
        in_specs=[half_spec, half_spec, half_spec, half_spec, dspec, dspec],
        out_specs=[half_spec, half_spec, half_spec, half_spec],
        out_shape=[half, half, half, half],
    )
    return f(o0lo, o0hi, o1lo, o1hi, d0, d1)


def _final_body(w0lo, w0hi, w1lo, w1hi, d0_ref, d1_ref,
                W0_ref, b0_ref, W1_ref, b1_ref, a_ref, Wd_ref, bd_ref,
                out_ref):
    h0 = jnp.concatenate([w0lo[...], w0hi[...]], axis=1) * d0_ref[...]
    h1 = jnp.concatenate([w1lo[...], w1hi[...]], axis=1) * d1_ref[...]
    e0 = jnp.dot(h0, W0_ref[...], preferred_element_type=jnp.float32,
                 precision=lax.Precision.HIGHEST) + b0_ref[...]
    e1 = jnp.dot(h1, W1_ref[...], preferred_element_type=jnp.float32,
                 precision=lax.Precision.HIGHEST) + b1_ref[...]
    a0 = a_ref[0, 0]
    a1 = a_ref[0, 1]
    m = jnp.maximum(a0, a1)
    x0 = jnp.exp(a0 - m)
    x1 = jnp.exp(a1 - m)
    ws0 = x0 / (x0 + x1)
    ws1 = x1 / (x0 + x1)
    fused = ws0 * e0 + ws1 * e1
    out_ref[...] = jnp.dot(fused, Wd_ref[...], preferred_element_type=jnp.float32,
                           precision=lax.Precision.HIGHEST) + bd_ref[...]


def _final_call(w0lo, w0hi, w1lo, w1hi, d0, d1, W0, b0, W1, b1, a2d, Wd, bd):
    grid = (_N // _BN,)
    half_spec = pl.BlockSpec((_BN, _HALF), lambda i: (i, 0))
    dspec = pl.BlockSpec((_BN, 1), lambda i: (i, 0))
    wspec = pl.BlockSpec((_D, _D), lambda i: (0, 0))
    bspec = pl.BlockSpec((1, _D), lambda i: (0, 0))
    f = pl.pallas_call(
        _final_body,
        grid=grid,
        in_specs=[
            half_spec, half_spec, half_spec, half_spec, dspec, dspec,
            wspec, bspec, wspec, bspec,
            pl.BlockSpec((1, 2), lambda i: (0, 0)),
            wspec, bspec,
        ],
        out_specs=pl.BlockSpec((_BN, _D), lambda i: (i, 0)),
        out_shape=jax.ShapeDtypeStruct((_N, _D), jnp.float32),
    )
    return f(w0lo, w0hi, w1lo, w1hi, d0, d1, W0, b0, W1, b1, a2d, Wd, bd)


def _pad_idx(idx, fill):
    pad = jnp.full((_E_PAD - _E,), fill, dtype=jnp.int32)
    return jnp.concatenate([idx.astype(jnp.int32), pad]).reshape(_E_PAD // 128, 128)


@jax.jit
def kernel(x, edge_index_0, edge_index_1, W_enc0, b_enc0, W_enc1, b_enc1,
           a, W_dec, b_dec):
    r0 = _pad_idx(edge_index_0[0], _N)
    c0 = _pad_idx(edge_index_0[1], 0)
    r1 = _pad_idx(edge_index_1[0], _N)
    c1 = _pad_idx(edge_index_1[1], 0)
    c0d = _pad_idx(edge_index_0[1], _N)
    c1d = _pad_idx(edge_index_1[1], _N)

    hp0, hp1 = _deg_call(c0d, c1d)
    hp0 = jnp.transpose(hp0, (1, 2, 0)).reshape(_N, _NS)
    hp1 = jnp.transpose(hp1, (1, 2, 0)).reshape(_N, _NS)

    y0lo, y0hi, y1lo, y1hi, d0, d1 = _prep_call(hp0, hp1, x)

    g0lo, g0hi, g1lo, g1hi = _prop_call(y0lo, y0hi, y1lo, y1hi, r0, c0, r1, c1)

    z0lo, z0hi, z1lo, z1hi = _mid_call(g0lo, g0hi, g1lo, g1hi, d0, d1)

    w0lo, w0hi, w1lo, w1hi = _prop_call(z0lo, z0hi, z1lo, z1hi, r0, c0, r1, c1)

    a2d = a.reshape(1, 2).astype(jnp.float32)
    return _final_call(w0lo, w0hi, w1lo, w1hi, d0, d1,
                       W_enc0, b_enc0.reshape(1, _D),
                       W_enc1, b_enc1.reshape(1, _D),
                       a2d, W_dec, b_dec.reshape(1, _D))


# speedup vs baseline: 1.2005x; 1.2005x over previous
import functools

import jax
import jax.numpy as jnp
from jax import lax
from jax.experimental import pallas as pl
from jax.experimental.pallas import tpu as pltpu
from jax.experimental.pallas import tpu_sc as plsc

_N = 10000
_D = 128
_E = 320000

_NC = 2
_NS = 16
_HALF = _D // 2

_CHUNK = 512
_IDX_ROWS = _CHUNK // 128
_EPT = 20480
_E_PAD = _EPT * _NS
_STEPS = _EPT // _CHUNK
_NPT = _N // _NS
_ACC_ROWS = _N + 16

_HIST_ROWS = _N // 16
_HIST_PAD = _HIST_ROWS + 8

_BN = 2000


def _deg_body(c0_2d, c1_2d, h0_out, h1_out, colbuf, hist):
    c = lax.axis_index("c")
    s = lax.axis_index("s")
    ones = jnp.full((16,), 1.0, dtype=jnp.float32)

    def run(col2d, hout):
        def zero_row(i, carry):
            hist[i] = jnp.zeros((16,), dtype=jnp.float32)
            return carry
        lax.fori_loop(0, _HIST_PAD, zero_row, 0)

        def step(t, carry):
            base = s * (_EPT // 128) + t * 16
            pltpu.sync_copy(col2d.at[pl.ds(base, 16)], colbuf)
            for j in range(16):
                for k in range(8):
                    v = colbuf[j, pl.ds(k * 16, 16)]
                    r = lax.shift_right_logical(v, 4)
                    cc = lax.bitwise_and(v, 15)
                    plsc.addupdate_scatter(hist, [r, cc], ones)
            return carry
        lax.fori_loop(0, _EPT // (16 * 128), step, 0)
        pltpu.sync_copy(hist.at[pl.ds(0, _HIST_ROWS)], hout.at[s])

    @pl.when(c == 0)
    def _():
        run(c0_2d, h0_out)

    @pl.when(c == 1)
    def _():
        run(c1_2d, h1_out)


def _deg_call(c0_2d, c1_2d):
    mesh = plsc.VectorSubcoreMesh(core_axis_name="c", subcore_axis_name="s", num_cores=_NC, num_subcores=_NS)
    f = pl.kernel(
        _deg_body,
        out_type=[
            jax.ShapeDtypeStruct((_NS, _HIST_ROWS, 16), jnp.float32),
            jax.ShapeDtypeStruct((_NS, _HIST_ROWS, 16), jnp.float32),
        ],
        mesh=mesh,
        scratch_types=[
            pltpu.VMEM((16, 128), jnp.int32),
            pltpu.VMEM((_HIST_PAD, 16), jnp.float32),
        ],
        compiler_params=pltpu.CompilerParams(needs_layout_passes=False),
    )
    return f(c0_2d, c1_2d)


def _prop_body(y0lo, y0hi, y1lo, y1hi, r0_2d, c0_2d, r1_2d, c1_2d,
               d0bc, d1bc,
               o0lo, o0hi, o1lo, o1hi,
               colbuf, rowbuf, rowsbuf, dbuf, ysp, acc, gsem, ssem):
    c = lax.axis_index("c")
    s = lax.axis_index("s")

    def one_pass(r2d, c2d):
        def step(t, carry):
            base = s * (_EPT // 128) + t * _IDX_ROWS
            pltpu.sync_copy(c2d.at[pl.ds(base, _IDX_ROWS)], colbuf)
            pltpu.sync_copy(r2d.at[pl.ds(base, _IDX_ROWS)], rowbuf)
            gh = [
                pltpu.async_copy(ysp.at[colbuf.at[j]],
                                 rowsbuf.at[pl.ds(j * 128, 128)], gsem)
                for j in range(_IDX_ROWS)
            ]
            for h in gh:
                h.wait()
            sh = [
                pltpu.async_copy(rowsbuf.at[pl.ds(j * 128, 128)],
                                 acc.at[rowbuf.at[j]], ssem, add=True)
                for j in range(_IDX_ROWS)
            ]
            for h in sh:
                h.wait()
            return carry

        lax.fori_loop(0, _STEPS, step, 0)

    def run_rel(y_hbm, dbc_hbm, out_hbm, r2d, c2d):
        pltpu.sync_copy(y_hbm.at[pl.ds(s * _NPT, _NPT)],
                        ysp.at[pl.ds(s * _NPT, _NPT)])
        pltpu.sync_copy(y_hbm.at[pl.ds(s * _NPT, _NPT)],
                        acc.at[pl.ds(s * _NPT, _NPT)])
        pltpu.sync_copy(dbc_hbm.at[pl.ds(s * _NPT, _NPT)], dbuf)
        plsc.subcore_barrier()

        one_pass(r2d, c2d)
        plsc.subcore_barrier()

        for c0, ln in ((0, _CHUNK), (_CHUNK, _NPT - _CHUNK)):
            pltpu.sync_copy(acc.at[pl.ds(s * _NPT + c0, ln)],
                            rowsbuf.at[pl.ds(0, ln)])

            def scale_row(i, carry):
                m = dbuf[c0 + i]
                for k in range(_HALF // 16):
                    sl = pl.ds(k * 16, 16)
                    rowsbuf[i, sl] = rowsbuf[i, sl] * m
                return carry

            lax.fori_loop(0, ln, scale_row, 0)
            pltpu.sync_copy(rowsbuf.at[pl.ds(0, ln)],
                            ysp.at[pl.ds(s * _NPT + c0, ln)])
            pltpu.sync_copy(rowsbuf.at[pl.ds(0, ln)],
                            acc.at[pl.ds(s * _NPT + c0, ln)])
        plsc.subcore_barrier()

        one_pass(r2d, c2d)
        plsc.subcore_barrier()
        pltpu.sync_copy(acc.at[pl.ds(s * _NPT, _NPT)],
                        out_hbm.at[pl.ds(s * _NPT, _NPT)])

    @pl.when(c == 0)
    def _():
        run_rel(y0lo, d0bc, o0lo, r0_2d, c0_2d)
        run_rel(y1lo, d1bc, o1lo, r1_2d, c1_2d)

    @pl.when(c == 1)
    def _():
        run_rel(y0hi, d0bc, o0hi, r0_2d, c0_2d)
        run_rel(y1hi, d1bc, o1hi, r1_2d, c1_2d)


def _prop_call(y0lo, y0hi, y1lo, y1hi, r0_2d, c0_2d, r1_2d, c1_2d, d0bc, d1bc):
    mesh = plsc.VectorSubcoreMesh(core_axis_name="c", subcore_axis_name="s", num_cores=_NC, num_subcores=_NS)
    half = jax.ShapeDtypeStruct((_N, _HALF), jnp.float32)
    f = pl.kernel(
        _prop_body,
        out_type=[half, half, half, half],
        mesh=mesh,
        scratch_types=[
            pltpu.VMEM((_IDX_ROWS, 128), jnp.int32),
            pltpu.VMEM((_IDX_ROWS, 128), jnp.int32),
            pltpu.VMEM((_CHUNK, _HALF), jnp.float32),
            pltpu.VMEM((_NPT, 16), jnp.float32),
            pltpu.VMEM_SHARED((_N, _HALF), jnp.float32),
            pltpu.VMEM_SHARED((_ACC_ROWS, _HALF), jnp.float32),
            pltpu.SemaphoreType.DMA,
            pltpu.SemaphoreType.DMA,
        ],
        compiler_params=pltpu.CompilerParams(use_tc_tiling_on_sc=False),
    )
    return f(y0lo, y0hi, y1lo, y1hi, r0_2d, c0_2d, r1_2d, c1_2d, d0bc, d1bc)


def _prep_body(h0_ref, h1_ref, x_ref,
               y0lo, y0hi, y1lo, y1hi, d0_ref, d1_ref, d0bc_ref, d1bc_ref):
    deg0 = 1.0 + jnp.sum(h0_ref[...], axis=1, keepdims=True)
    deg1 = 1.0 + jnp.sum(h1_ref[...], axis=1, keepdims=True)
    dinv0 = jnp.where(deg0 > 0, lax.rsqrt(deg0), 0.0)
    dinv1 = jnp.where(deg1 > 0, lax.rsqrt(deg1), 0.0)
    x = x_ref[...]
    y0 = x * dinv0
    y1 = x * dinv1
    y0lo[...] = y0[:, :_HALF]
    y0hi[...] = y0[:, _HALF:]
    y1lo[...] = y1[:, :_HALF]
    y1hi[...] = y1[:, _HALF:]
    d0_ref[...] = dinv0
    d1_ref[...] = dinv1
    d0bc_ref[...] = jnp.broadcast_to(dinv0 * dinv0, (_BN, 16))
    d1bc_ref[...] = jnp.broadcast_to(dinv1 * dinv1, (_BN, 16))


def _prep_call(hp0, hp1, x):
    grid = (_N // _BN,)
    half_spec = pl.BlockSpec((_BN, _HALF), lambda i: (i, 0))
    f = pl.pallas_call(
        _prep_body,
        grid=grid,
        in_specs=[
            pl.BlockSpec((_BN, 16), lambda i: (i, 0)),
            pl.BlockSpec((_BN, 16), lambda i: (i, 0)),
            pl.BlockSpec((_BN, _D), lambda i: (i, 0)),
        ],
        out_specs=[
            half_spec, half_spec, half_spec, half_spec,
            pl.BlockSpec((_BN, 1), lambda i: (i, 0)),
            pl.BlockSpec((_BN, 1), lambda i: (i, 0)),
            pl.BlockSpec((_BN, 16), lambda i: (i, 0)),
            pl.BlockSpec((_BN, 16), lambda i: (i, 0)),
        ],
        out_shape=[
            jax.ShapeDtypeStruct((_N, _HALF), jnp.float32),
            jax.ShapeDtypeStruct((_N, _HALF), jnp.float32),
            jax.ShapeDtypeStruct((_N, _HALF), jnp.float32),
            jax.ShapeDtypeStruct((_N, _HALF), jnp.float32),
            jax.ShapeDtypeStruct((_N, 1), jnp.float32),
            jax.ShapeDtypeStruct((_N, 1), jnp.float32),
            jax.ShapeDtypeStruct((_N, 16), jnp.float32),
            jax.ShapeDtypeStruct((_N, 16), jnp.float32),
        ],
    )
    return f(hp0, hp1, x)


def _final_body(w0lo, w0hi, w1lo, w1hi, d0_ref, d1_ref,
                W0_ref, b0_ref, W1_ref, b1_ref, a_ref, Wd_ref, bd_ref,
                out_ref):
    h0 = jnp.concatenate([w0lo[...], w0hi[...]], axis=1) * d0_ref[...]
    h1 = jnp.concatenate([w1lo[...], w1hi[...]], axis=1) * d1_ref[...]
    e0 = jnp.dot(h0, W0_ref[...], preferred_element_type=jnp.float32,
                 precision=lax.Precision.HIGHEST) + b0_ref[...]
    e1 = jnp.dot(h1, W1_ref[...], preferred_element_type=jnp.float32,
                 precision=lax.Precision.HIGHEST) + b1_ref[...]
    a0 = a_ref[0, 0]
    a1 = a_ref[0, 1]
    m = jnp.maximum(a0, a1)
    x0 = jnp.exp(a0 - m)
    x1 = jnp.exp(a1 - m)
    ws0 = x0 / (x0 + x1)
    ws1 = x1 / (x0 + x1)
    fused = ws0 * e0 + ws1 * e1
    out_ref[...] = jnp.dot(fused, Wd_ref[...], preferred_element_type=jnp.float32,
                           precision=lax.Precision.HIGHEST) + bd_ref[...]


def _final_call(w0lo, w0hi, w1lo, w1hi, d0, d1, W0, b0, W1, b1, a2d, Wd, bd):
    grid = (_N // _BN,)
    half_spec = pl.BlockSpec((_BN, _HALF), lambda i: (i, 0))
    dspec = pl.BlockSpec((_BN, 1), lambda i: (i, 0))
    wspec = pl.BlockSpec((_D, _D), lambda i: (0, 0))
    bspec = pl.BlockSpec((1, _D), lambda i: (0, 0))
    f = pl.pallas_call(
        _final_body,
        grid=grid,
        in_specs=[
            half_spec, half_spec, half_spec, half_spec, dspec, dspec,
            wspec, bspec, wspec, bspec,
            pl.BlockSpec((1, 2), lambda i: (0, 0)),
            wspec, bspec,
        ],
        out_specs=pl.BlockSpec((_BN, _D), lambda i: (i, 0)),
        out_shape=jax.ShapeDtypeStruct((_N, _D), jnp.float32),
    )
    return f(w0lo, w0hi, w1lo, w1hi, d0, d1, W0, b0, W1, b1, a2d, Wd, bd)


def _pad_idx(idx, fill):
    pad = jnp.full((_E_PAD - _E,), fill, dtype=jnp.int32)
    return jnp.concatenate([idx.astype(jnp.int32), pad]).reshape(_E_PAD // 128, 128)


@jax.jit
def kernel(x, edge_index_0, edge_index_1, W_enc0, b_enc0, W_enc1, b_enc1,
           a, W_dec, b_dec):
    r0 = _pad_idx(edge_index_0[0], _N)
    c0 = _pad_idx(edge_index_0[1], 0)
    r1 = _pad_idx(edge_index_1[0], _N)
    c1 = _pad_idx(edge_index_1[1], 0)
    c0d = _pad_idx(edge_index_0[1], _N)
    c1d = _pad_idx(edge_index_1[1], _N)

    hp0, hp1 = _deg_call(c0d, c1d)
    hp0 = jnp.transpose(hp0, (1, 2, 0)).reshape(_N, _NS)
    hp1 = jnp.transpose(hp1, (1, 2, 0)).reshape(_N, _NS)

    y0lo, y0hi, y1lo, y1hi, d0, d1, d0bc, d1bc = _prep_call(hp0, hp1, x)

    w0lo, w0hi, w1lo, w1hi = _prop_call(y0lo, y0hi, y1lo, y1hi,
                                        r0, c0, r1, c1, d0bc, d1bc)

    a2d = a.reshape(1, 2).astype(jnp.float32)
    return _final_call(w0lo, w0hi, w1lo, w1hi, d0, d1,
                       W_enc0, b_enc0.reshape(1, _D),
                       W_enc1, b_enc1.reshape(1, _D),
                       a2d, W_dec, b_dec.reshape(1, _D))


# speedup vs baseline: 1.2170x; 1.0137x over previous
import functools

import jax
import jax.numpy as jnp
from jax import lax
from jax.experimental import pallas as pl
from jax.experimental.pallas import tpu as pltpu
from jax.experimental.pallas import tpu_sc as plsc

_N = 10000
_D = 128
_E = 320000

_NC = 2
_NS = 16
_HALF = _D // 2

_CHUNK = 512
_IDX_ROWS = _CHUNK // 128
_EPT = 20480
_E_PAD = _EPT * _NS
_STEPS = _EPT // _CHUNK
_E_PAD2 = _E_PAD + _CHUNK
_NPT = _N // _NS
_ACC_ROWS = _N + 16

_HIST_ROWS = _N // 16
_HIST_PAD = _HIST_ROWS + 8

_BN = 2000


def _deg_body(c0_2d, c1_2d, h0_out, h1_out, colbuf, hist):
    c = lax.axis_index("c")
    s = lax.axis_index("s")
    ones = jnp.full((16,), 1.0, dtype=jnp.float32)

    def run(col2d, hout):
        def zero_row(i, carry):
            hist[i] = jnp.zeros((16,), dtype=jnp.float32)
            return carry
        lax.fori_loop(0, _HIST_PAD, zero_row, 0)

        def step(t, carry):
            base = s * (_EPT // 128) + t * 16
            pltpu.sync_copy(col2d.at[pl.ds(base, 16)], colbuf)
            for j in range(16):
                for k in range(8):
                    v = colbuf[j, pl.ds(k * 16, 16)]
                    r = lax.shift_right_logical(v, 4)
                    cc = lax.bitwise_and(v, 15)
                    plsc.addupdate_scatter(hist, [r, cc], ones)
            return carry
        lax.fori_loop(0, _EPT // (16 * 128), step, 0)
        pltpu.sync_copy(hist.at[pl.ds(0, _HIST_ROWS)], hout.at[s])

    @pl.when(c == 0)
    def _():
        run(c0_2d, h0_out)

    @pl.when(c == 1)
    def _():
        run(c1_2d, h1_out)


def _deg_call(c0_2d, c1_2d):
    mesh = plsc.VectorSubcoreMesh(core_axis_name="c", subcore_axis_name="s", num_cores=_NC, num_subcores=_NS)
    f = pl.kernel(
        _deg_body,
        out_type=[
            jax.ShapeDtypeStruct((_NS, _HIST_ROWS, 16), jnp.float32),
            jax.ShapeDtypeStruct((_NS, _HIST_ROWS, 16), jnp.float32),
        ],
        mesh=mesh,
        scratch_types=[
            pltpu.VMEM((16, 128), jnp.int32),
            pltpu.VMEM((_HIST_PAD, 16), jnp.float32),
        ],
        compiler_params=pltpu.CompilerParams(needs_layout_passes=False),
    )
    return f(c0_2d, c1_2d)


def _prop_body(y0lo, y0hi, y1lo, y1hi, r0_2d, c0_2d, r1_2d, c1_2d,
               d0bc, d1bc,
               o0lo, o0hi, o1lo, o1hi,
               cbuf0, cbuf1, rbuf0, rbuf1, rowsbuf, dbuf, ysp, acc,
               gsem, ssem, isem0, isem1):
    c = lax.axis_index("c")
    s = lax.axis_index("s")

    def one_pass(r2d, c2d):
        def idx_issue(t, cb, rb, sem):
            base = s * (_EPT // 128) + t * _IDX_ROWS
            pltpu.async_copy(c2d.at[pl.ds(base, _IDX_ROWS)], cb, sem)
            pltpu.async_copy(r2d.at[pl.ds(base, _IDX_ROWS)], rb, sem)

        def idx_wait(cb, rb, sem):
            pltpu.make_async_copy(c2d.at[pl.ds(0, _IDX_ROWS)], cb, sem).wait()
            pltpu.make_async_copy(r2d.at[pl.ds(0, _IDX_ROWS)], rb, sem).wait()

        def chunk(cb, rb):
            gh = [
                pltpu.async_copy(ysp.at[cb.at[j]],
                                 rowsbuf.at[pl.ds(j * 128, 128)], gsem)
                for j in range(_IDX_ROWS)
            ]
            for h in gh:
                h.wait()
            sh = [
                pltpu.async_copy(rowsbuf.at[pl.ds(j * 128, 128)],
                                 acc.at[rb.at[j]], ssem, add=True)
                for j in range(_IDX_ROWS)
            ]
            for h in sh:
                h.wait()

        idx_issue(0, cbuf0, rbuf0, isem0)

        def group(g, carry):
            t = 2 * g
            idx_issue(t + 1, cbuf1, rbuf1, isem1)
            idx_wait(cbuf0, rbuf0, isem0)
            chunk(cbuf0, rbuf0)
            idx_issue(t + 2, cbuf0, rbuf0, isem0)
            idx_wait(cbuf1, rbuf1, isem1)
            chunk(cbuf1, rbuf1)
            return carry

        lax.fori_loop(0, _STEPS // 2, group, 0)
        idx_wait(cbuf0, rbuf0, isem0)

    def run_rel(y_hbm, dbc_hbm, out_hbm, r2d, c2d):
        pltpu.sync_copy(y_hbm.at[pl.ds(s * _NPT, _NPT)],
                        ysp.at[pl.ds(s * _NPT, _NPT)])
        pltpu.sync_copy(y_hbm.at[pl.ds(s * _NPT, _NPT)],
                        acc.at[pl.ds(s * _NPT, _NPT)])
        pltpu.sync_copy(dbc_hbm.at[pl.ds(s * _NPT, _NPT)], dbuf)
        plsc.subcore_barrier()

        one_pass(r2d, c2d)
        plsc.subcore_barrier()

        for c0, ln in ((0, _CHUNK), (_CHUNK, _NPT - _CHUNK)):
            pltpu.sync_copy(acc.at[pl.ds(s * _NPT + c0, ln)],
                            rowsbuf.at[pl.ds(0, ln)])

            def scale_row(i, carry):
                m = dbuf[c0 + i]
                for k in range(_HALF // 16):
                    sl = pl.ds(k * 16, 16)
                    rowsbuf[i, sl] = rowsbuf[i, sl] * m
                return carry

            lax.fori_loop(0, ln, scale_row, 0)
            pltpu.sync_copy(rowsbuf.at[pl.ds(0, ln)],
                            ysp.at[pl.ds(s * _NPT + c0, ln)])
            pltpu.sync_copy(rowsbuf.at[pl.ds(0, ln)],
                            acc.at[pl.ds(s * _NPT + c0, ln)])
        plsc.subcore_barrier()

        one_pass(r2d, c2d)
        plsc.subcore_barrier()
        pltpu.sync_copy(acc.at[pl.ds(s * _NPT, _NPT)],
                        out_hbm.at[pl.ds(s * _NPT, _NPT)])

    @pl.when(c == 0)
    def _():
        run_rel(y0lo, d0bc, o0lo, r0_2d, c0_2d)
        run_rel(y1lo, d1bc, o1lo, r1_2d, c1_2d)

    @pl.when(c == 1)
    def _():
        run_rel(y0hi, d0bc, o0hi, r0_2d, c0_2d)
        run_rel(y1hi, d1bc, o1hi, r1_2d, c1_2d)


def _prop_call(y0lo, y0hi, y1lo, y1hi, r0_2d, c0_2d, r1_2d, c1_2d, d0bc, d1bc):
    mesh = plsc.VectorSubcoreMesh(core_axis_name="c", subcore_axis_name="s", num_cores=_NC, num_subcores=_NS)
    half = jax.ShapeDtypeStruct((_N, _HALF), jnp.float32)
    f = pl.kernel(
        _prop_body,
        out_type=[half, half, half, half],
        mesh=mesh,
        scratch_types=[
            pltpu.VMEM((_IDX_ROWS, 128), jnp.int32),
            pltpu.VMEM((_IDX_ROWS, 128), jnp.int32),
            pltpu.VMEM((_IDX_ROWS, 128), jnp.int32),
            pltpu.VMEM((_IDX_ROWS, 128), jnp.int32),
            pltpu.VMEM((_CHUNK, _HALF), jnp.float32),
            pltpu.VMEM((_NPT, 16), jnp.float32),
            pltpu.VMEM_SHARED((_N, _HALF), jnp.float32),
            pltpu.VMEM_SHARED((_ACC_ROWS, _HALF), jnp.float32),
            pltpu.SemaphoreType.DMA,
            pltpu.SemaphoreType.DMA,
            pltpu.SemaphoreType.DMA,
            pltpu.SemaphoreType.DMA,
        ],
        compiler_params=pltpu.CompilerParams(use_tc_tiling_on_sc=False),
    )
    return f(y0lo, y0hi, y1lo, y1hi, r0_2d, c0_2d, r1_2d, c1_2d, d0bc, d1bc)


def _prep_body(h0_ref, h1_ref, x_ref,
               y0lo, y0hi, y1lo, y1hi, d0_ref, d1_ref, d0bc_ref, d1bc_ref):
    deg0 = 1.0 + jnp.sum(h0_ref[...], axis=1, keepdims=True)
    deg1 = 1.0 + jnp.sum(h1_ref[...], axis=1, keepdims=True)
    dinv0 = jnp.where(deg0 > 0, lax.rsqrt(deg0), 0.0)
    dinv1 = jnp.where(deg1 > 0, lax.rsqrt(deg1), 0.0)
    x = x_ref[...]
    y0 = x * dinv0
    y1 = x * dinv1
    y0lo[...] = y0[:, :_HALF]
    y0hi[...] = y0[:, _HALF:]
    y1lo[...] = y1[:, :_HALF]
    y1hi[...] = y1[:, _HALF:]
    d0_ref[...] = dinv0
    d1_ref[...] = dinv1
    d0bc_ref[...] = jnp.broadcast_to(dinv0 * dinv0, (_BN, 16))
    d1bc_ref[...] = jnp.broadcast_to(dinv1 * dinv1, (_BN, 16))


def _prep_call(hp0, hp1, x):
    grid = (_N // _BN,)
    half_spec = pl.BlockSpec((_BN, _HALF), lambda i: (i, 0))
    f = pl.pallas_call(
        _prep_body,
        grid=grid,
        in_specs=[
            pl.BlockSpec((_BN, 16), lambda i: (i, 0)),
            pl.BlockSpec((_BN, 16), lambda i: (i, 0)),
            pl.BlockSpec((_BN, _D), lambda i: (i, 0)),
        ],
        out_specs=[
            half_spec, half_spec, half_spec, half_spec,
            pl.BlockSpec((_BN, 1), lambda i: (i, 0)),
            pl.BlockSpec((_BN, 1), lambda i: (i, 0)),
            pl.BlockSpec((_BN, 16), lambda i: (i, 0)),
            pl.BlockSpec((_BN, 16), lambda i: (i, 0)),
        ],
        out_shape=[
            jax.ShapeDtypeStruct((_N, _HALF), jnp.float32),
            jax.ShapeDtypeStruct((_N, _HALF), jnp.float32),
            jax.ShapeDtypeStruct((_N, _HALF), jnp.float32),
            jax.ShapeDtypeStruct((_N, _HALF), jnp.float32),
            jax.ShapeDtypeStruct((_N, 1), jnp.float32),
            jax.ShapeDtypeStruct((_N, 1), jnp.float32),
            jax.ShapeDtypeStruct((_N, 16), jnp.float32),
            jax.ShapeDtypeStruct((_N, 16), jnp.float32),
        ],
    )
    return f(hp0, hp1, x)


def _final_body(w0lo, w0hi, w1lo, w1hi, d0_ref, d1_ref,
                W0_ref, b0_ref, W1_ref, b1_ref, a_ref, Wd_ref, bd_ref,
                out_ref):
    h0 = jnp.concatenate([w0lo[...], w0hi[...]], axis=1) * d0_ref[...]
    h1 = jnp.concatenate([w1lo[...], w1hi[...]], axis=1) * d1_ref[...]
    e0 = jnp.dot(h0, W0_ref[...], preferred_element_type=jnp.float32,
                 precision=lax.Precision.HIGHEST) + b0_ref[...]
    e1 = jnp.dot(h1, W1_ref[...], preferred_element_type=jnp.float32,
                 precision=lax.Precision.HIGHEST) + b1_ref[...]
    a0 = a_ref[0, 0]
    a1 = a_ref[0, 1]
    m = jnp.maximum(a0, a1)
    x0 = jnp.exp(a0 - m)
    x1 = jnp.exp(a1 - m)
    ws0 = x0 / (x0 + x1)
    ws1 = x1 / (x0 + x1)
    fused = ws0 * e0 + ws1 * e1
    out_ref[...] = jnp.dot(fused, Wd_ref[...], preferred_element_type=jnp.float32,
                           precision=lax.Precision.HIGHEST) + bd_ref[...]


def _final_call(w0lo, w0hi, w1lo, w1hi, d0, d1, W0, b0, W1, b1, a2d, Wd, bd):
    grid = (_N // _BN,)
    half_spec = pl.BlockSpec((_BN, _HALF), lambda i: (i, 0))
    dspec = pl.BlockSpec((_BN, 1), lambda i: (i, 0))
    wspec = pl.BlockSpec((_D, _D), lambda i: (0, 0))
    bspec = pl.BlockSpec((1, _D), lambda i: (0, 0))
    f = pl.pallas_call(
        _final_body,
        grid=grid,
        in_specs=[
            half_spec, half_spec, half_spec, half_spec, dspec, dspec,
            wspec, bspec, wspec, bspec,
            pl.BlockSpec((1, 2), lambda i: (0, 0)),
            wspec, bspec,
        ],
        out_specs=pl.BlockSpec((_BN, _D), lambda i: (i, 0)),
        out_shape=jax.ShapeDtypeStruct((_N, _D), jnp.float32),
    )
    return f(w0lo, w0hi, w1lo, w1hi, d0, d1, W0, b0, W1, b1, a2d, Wd, bd)


def _pad_idx(idx, fill):
    pad = jnp.full((_E_PAD2 - _E,), fill, dtype=jnp.int32)
    return jnp.concatenate([idx.astype(jnp.int32), pad]).reshape(_E_PAD2 // 128, 128)


@jax.jit
def kernel(x, edge_index_0, edge_index_1, W_enc0, b_enc0, W_enc1, b_enc1,
           a, W_dec, b_dec):
    r0 = _pad_idx(edge_index_0[0], _N)
    c0 = _pad_idx(edge_index_0[1], 0)
    r1 = _pad_idx(edge_index_1[0], _N)
    c1 = _pad_idx(edge_index_1[1], 0)
    c0d = _pad_idx(edge_index_0[1], _N)
    c1d = _pad_idx(edge_index_1[1], _N)

    hp0, hp1 = _deg_call(c0d, c1d)
    hp0 = jnp.transpose(hp0, (1, 2, 0)).reshape(_N, _NS)
    hp1 = jnp.transpose(hp1, (1, 2, 0)).reshape(_N, _NS)

    y0lo, y0hi, y1lo, y1hi, d0, d1, d0bc, d1bc = _prep_call(hp0, hp1, x)

    w0lo, w0hi, w1lo, w1hi = _prop_call(y0lo, y0hi, y1lo, y1hi,
                                        r0, c0, r1, c1, d0bc, d1bc)

    a2d = a.reshape(1, 2).astype(jnp.float32)
    return _final_call(w0lo, w0hi, w1lo, w1hi, d0, d1,
                       W_enc0, b_enc0.reshape(1, _D),
                       W_enc1, b_enc1.reshape(1, _D),
                       a2d, W_dec, b_dec.reshape(1, _D))


# speedup vs baseline: 1.3546x; 1.1131x over previous
import functools

import jax
import jax.numpy as jnp
from jax import lax
from jax.experimental import pallas as pl
from jax.experimental.pallas import tpu as pltpu
from jax.experimental.pallas import tpu_sc as plsc

_N = 10000
_D = 128
_E = 320000

_NC = 2
_NS = 16
_HALF = _D // 2

_CHUNK = 512
_IDX_ROWS = _CHUNK // 128
_ER = 2 * _E // 256
_RPT = _ER // _NS
_STEPS = _RPT // _IDX_ROWS
_XTRA = (_ER - _RPT * _NS) // _IDX_ROWS
_NPT = _N // _NS

_HIST_ROWS = _N // 16

_BN = 2000


def _deg_body(e0_2d, e1_2d, h0_out, h1_out, colbuf, hist):
    c = lax.axis_index("c")
    s = lax.axis_index("s")
    ones = jnp.full((16,), 1.0, dtype=jnp.float32)

    def run(e2d, hout):
        def zero_row(i, carry):
            hist[i] = jnp.zeros((16,), dtype=jnp.float32)
            return carry
        lax.fori_loop(0, _HIST_ROWS, zero_row, 0)

        def step(t, carry):
            base = _ER + _RPT * s + t * _IDX_ROWS
            pltpu.sync_copy(e2d.at[pl.ds(base, _IDX_ROWS)], colbuf)
            for j in range(_IDX_ROWS):
                for k in range(8):
                    v = colbuf[j, pl.ds(k * 16, 16)]
                    r = lax.shift_right_logical(v, 4)
                    cc = lax.bitwise_and(v, 15)
                    plsc.addupdate_scatter(hist, [r, cc], ones)
            return carry
        lax.fori_loop(0, _STEPS, step, 0)

        @pl.when(s == _NS - 1)
        def _():
            lax.fori_loop(_STEPS, _STEPS + _XTRA, step, 0)

        pltpu.sync_copy(hist.at[pl.ds(0, _HIST_ROWS)], hout.at[s])

    @pl.when(c == 0)
    def _():
        run(e0_2d, h0_out)

    @pl.when(c == 1)
    def _():
        run(e1_2d, h1_out)


def _deg_call(e0_2d, e1_2d):
    mesh = plsc.VectorSubcoreMesh(core_axis_name="c", subcore_axis_name="s", num_cores=_NC, num_subcores=_NS)
    f = pl.kernel(
        _deg_body,
        out_type=[
            jax.ShapeDtypeStruct((_NS, _HIST_ROWS, 16), jnp.float32),
            jax.ShapeDtypeStruct((_NS, _HIST_ROWS, 16), jnp.float32),
        ],
        mesh=mesh,
        scratch_types=[
            pltpu.VMEM((_IDX_ROWS, 128), jnp.int32),
            pltpu.VMEM((_HIST_ROWS, 16), jnp.float32),
        ],
        compiler_params=pltpu.CompilerParams(needs_layout_passes=False),
    )
    return f(e0_2d, e1_2d)


def _prop_body(y0lo, y0hi, y1lo, y1hi, e0_2d, e1_2d,
               d0bc, d1bc,
               o0lo, o0hi, o1lo, o1hi,
               cbuf0, cbuf1, rbuf0, rbuf1, rowsbuf, dbuf, ysp, acc,
               gsem, ssem, isem0, isem1):
    c = lax.axis_index("c")
    s = lax.axis_index("s")

    def one_pass(e2d):
        def idx_issue(t, cb, rb, sem):
            base = jnp.minimum(_RPT * s + t * _IDX_ROWS, _ER - _IDX_ROWS)
            pltpu.async_copy(e2d.at[pl.ds(_ER + base, _IDX_ROWS)], cb, sem)
            pltpu.async_copy(e2d.at[pl.ds(base, _IDX_ROWS)], rb, sem)

        def idx_wait(cb, rb, sem):
            pltpu.make_async_copy(e2d.at[pl.ds(0, _IDX_ROWS)], cb, sem).wait()
            pltpu.make_async_copy(e2d.at[pl.ds(0, _IDX_ROWS)], rb, sem).wait()

        def chunk(cb, rb):
            gh = [
                pltpu.async_copy(ysp.at[cb.at[j]],
                                 rowsbuf.at[pl.ds(j * 128, 128)], gsem)
                for j in range(_IDX_ROWS)
            ]
            for h in gh:
                h.wait()
            sh = [
                pltpu.async_copy(rowsbuf.at[pl.ds(j * 128, 128)],
                                 acc.at[rb.at[j]], ssem, add=True)
                for j in range(_IDX_ROWS)
            ]
            for h in sh:
                h.wait()

        idx_issue(0, cbuf0, rbuf0, isem0)

        def group(g, carry):
            t = 2 * g
            idx_issue(t + 1, cbuf1, rbuf1, isem1)
            idx_wait(cbuf0, rbuf0, isem0)
            chunk(cbuf0, rbuf0)
            idx_issue(t + 2, cbuf0, rbuf0, isem0)
            idx_wait(cbuf1, rbuf1, isem1)
            chunk(cbuf1, rbuf1)
            return carry

        lax.fori_loop(0, (_STEPS - 1) // 2, group, 0)
        idx_wait(cbuf0, rbuf0, isem0)
        chunk(cbuf0, rbuf0)

        @pl.when(s == _NS - 1)
        def _():
            idx_issue(_STEPS, cbuf1, rbuf1, isem1)
            idx_wait(cbuf1, rbuf1, isem1)
            chunk(cbuf1, rbuf1)

    def run_rel(y_hbm, dbc_hbm, out_hbm, e2d):
        pltpu.sync_copy(y_hbm.at[pl.ds(s * _NPT, _NPT)],
                        ysp.at[pl.ds(s * _NPT, _NPT)])
        pltpu.sync_copy(y_hbm.at[pl.ds(s * _NPT, _NPT)],
                        acc.at[pl.ds(s * _NPT, _NPT)])
        pltpu.sync_copy(dbc_hbm.at[pl.ds(s * _NPT, _NPT)], dbuf)
        plsc.subcore_barrier()

        one_pass(e2d)
        plsc.subcore_barrier()

        for c0, ln in ((0, _CHUNK), (_CHUNK, _NPT - _CHUNK)):
            pltpu.sync_copy(acc.at[pl.ds(s * _NPT + c0, ln)],
                            rowsbuf.at[pl.ds(0, ln)])

            def scale_row(i, carry):
                m = dbuf[c0 + i]
                for k in range(_HALF // 16):
                    sl = pl.ds(k * 16, 16)
                    rowsbuf[i, sl] = rowsbuf[i, sl] * m
                return carry

            lax.fori_loop(0, ln, scale_row, 0)
            pltpu.sync_copy(rowsbuf.at[pl.ds(0, ln)],
                            ysp.at[pl.ds(s * _NPT + c0, ln)])
            pltpu.sync_copy(rowsbuf.at[pl.ds(0, ln)],
                            acc.at[pl.ds(s * _NPT + c0, ln)])
        plsc.subcore_barrier()

        one_pass(e2d)
        plsc.subcore_barrier()
        pltpu.sync_copy(acc.at[pl.ds(s * _NPT, _NPT)],
                        out_hbm.at[pl.ds(s * _NPT, _NPT)])

    @pl.when(c == 0)
    def _():
        run_rel(y0lo, d0bc, o0lo, e0_2d)
        run_rel(y1lo, d1bc, o1lo, e1_2d)

    @pl.when(c == 1)
    def _():
        run_rel(y0hi, d0bc, o0hi, e0_2d)
        run_rel(y1hi, d1bc, o1hi, e1_2d)


def _prop_call(y0lo, y0hi, y1lo, y1hi, e0_2d, e1_2d, d0bc, d1bc):
    mesh = plsc.VectorSubcoreMesh(core_axis_name="c", subcore_axis_name="s", num_cores=_NC, num_subcores=_NS)
    half = jax.ShapeDtypeStruct((_N, _HALF), jnp.float32)
    f = pl.kernel(
        _prop_body,
        out_type=[half, half, half, half],
        mesh=mesh,
        scratch_types=[
            pltpu.VMEM((_IDX_ROWS, 128), jnp.int32),
            pltpu.VMEM((_IDX_ROWS, 128), jnp.int32),
            pltpu.VMEM((_IDX_ROWS, 128), jnp.int32),
            pltpu.VMEM((_IDX_ROWS, 128), jnp.int32),
            pltpu.VMEM((_CHUNK, _HALF), jnp.float32),
            pltpu.VMEM((_NPT, 16), jnp.float32),
            pltpu.VMEM_SHARED((_N, _HALF), jnp.float32),
            pltpu.VMEM_SHARED((_N, _HALF), jnp.float32),
            pltpu.SemaphoreType.DMA,
            pltpu.SemaphoreType.DMA,
            pltpu.SemaphoreType.DMA,
            pltpu.SemaphoreType.DMA,
        ],
        compiler_params=pltpu.CompilerParams(use_tc_tiling_on_sc=False),
    )
    return f(y0lo, y0hi, y1lo, y1hi, e0_2d, e1_2d, d0bc, d1bc)


def _prep_body(h0_ref, h1_ref, x_ref,
               y0lo, y0hi, y1lo, y1hi, d0_ref, d1_ref, d0bc_ref, d1bc_ref):
    deg0 = 1.0 + jnp.sum(h0_ref[...], axis=1, keepdims=True)
    deg1 = 1.0 + jnp.sum(h1_ref[...], axis=1, keepdims=True)
    dinv0 = jnp.where(deg0 > 0, lax.rsqrt(deg0), 0.0)
    dinv1 = jnp.where(deg1 > 0, lax.rsqrt(deg1), 0.0)
    x = x_ref[...]
    y0 = x * dinv0
    y1 = x * dinv1
    y0lo[...] = y0[:, :_HALF]
    y0hi[...] = y0[:, _HALF:]
    y1lo[...] = y1[:, :_HALF]
    y1hi[...] = y1[:, _HALF:]
    d0_ref[...] = dinv0
    d1_ref[...] = dinv1
    d0bc_ref[...] = jnp.broadcast_to(dinv0 * dinv0, (_BN, 16))
    d1bc_ref[...] = jnp.broadcast_to(dinv1 * dinv1, (_BN, 16))


def _prep_call(hp0, hp1, x):
    grid = (_N // _BN,)
    half_spec = pl.BlockSpec((_BN, _HALF), lambda i: (i, 0))
    f = pl.pallas_call(
        _prep_body,
        grid=grid,
        in_specs=[
            pl.BlockSpec((_BN, 16), lambda i: (i, 0)),
            pl.BlockSpec((_BN, 16), lambda i: (i, 0)),
            pl.BlockSpec((_BN, _D), lambda i: (i, 0)),
        ],
        out_specs=[
            half_spec, half_spec, half_spec, half_spec,
            pl.BlockSpec((_BN, 1), lambda i: (i, 0)),
            pl.BlockSpec((_BN, 1), lambda i: (i, 0)),
            pl.BlockSpec((_BN, 16), lambda i: (i, 0)),
            pl.BlockSpec((_BN, 16), lambda i: (i, 0)),
        ],
        out_shape=[
            jax.ShapeDtypeStruct((_N, _HALF), jnp.float32),
            jax.ShapeDtypeStruct((_N, _HALF), jnp.float32),
            jax.ShapeDtypeStruct((_N, _HALF), jnp.float32),
            jax.ShapeDtypeStruct((_N, _HALF), jnp.float32),
            jax.ShapeDtypeStruct((_N, 1), jnp.float32),
            jax.ShapeDtypeStruct((_N, 1), jnp.float32),
            jax.ShapeDtypeStruct((_N, 16), jnp.float32),
            jax.ShapeDtypeStruct((_N, 16), jnp.float32),
        ],
    )
    return f(hp0, hp1, x)


def _final_body(w0lo, w0hi, w1lo, w1hi, d0_ref, d1_ref,
                W0_ref, b0_ref, W1_ref, b1_ref, a_ref, Wd_ref, bd_ref,
                out_ref):
    h0 = jnp.concatenate([w0lo[...], w0hi[...]], axis=1) * d0_ref[...]
    h1 = jnp.concatenate([w1lo[...], w1hi[...]], axis=1) * d1_ref[...]
    e0 = jnp.dot(h0, W0_ref[...], preferred_element_type=jnp.float32,
                 precision=lax.Precision.HIGHEST) + b0_ref[...]
    e1 = jnp.dot(h1, W1_ref[...], preferred_element_type=jnp.float32,
                 precision=lax.Precision.HIGHEST) + b1_ref[...]
    a0 = a_ref[0, 0]
    a1 = a_ref[0, 1]
    m = jnp.maximum(a0, a1)
    x0 = jnp.exp(a0 - m)
    x1 = jnp.exp(a1 - m)
    ws0 = x0 / (x0 + x1)
    ws1 = x1 / (x0 + x1)
    fused = ws0 * e0 + ws1 * e1
    out_ref[...] = jnp.dot(fused, Wd_ref[...], preferred_element_type=jnp.float32,
                           precision=lax.Precision.HIGHEST) + bd_ref[...]


def _final_call(w0lo, w0hi, w1lo, w1hi, d0, d1, W0, b0, W1, b1, a2d, Wd, bd):
    grid = (_N // _BN,)
    half_spec = pl.BlockSpec((_BN, _HALF), lambda i: (i, 0))
    dspec = pl.BlockSpec((_BN, 1), lambda i: (i, 0))
    wspec = pl.BlockSpec((_D, _D), lambda i: (0, 0))
    bspec = pl.BlockSpec((1, _D), lambda i: (0, 0))
    f = pl.pallas_call(
        _final_body,
        grid=grid,
        in_specs=[
            half_spec, half_spec, half_spec, half_spec, dspec, dspec,
            wspec, bspec, wspec, bspec,
            pl.BlockSpec((1, 2), lambda i: (0, 0)),
            wspec, bspec,
        ],
        out_specs=pl.BlockSpec((_BN, _D), lambda i: (i, 0)),
        out_shape=jax.ShapeDtypeStruct((_N, _D), jnp.float32),
    )
    return f(w0lo, w0hi, w1lo, w1hi, d0, d1, W0, b0, W1, b1, a2d, Wd, bd)


@jax.jit
def kernel(x, edge_index_0, edge_index_1, W_enc0, b_enc0, W_enc1, b_enc1,
           a, W_dec, b_dec):
    e0 = edge_index_0.astype(jnp.int32).reshape(2 * _ER, 128)
    e1 = edge_index_1.astype(jnp.int32).reshape(2 * _ER, 128)

    hp0, hp1 = _deg_call(e0, e1)
    hp0 = jnp.transpose(hp0, (1, 2, 0)).reshape(_N, _NS)
    hp1 = jnp.transpose(hp1, (1, 2, 0)).reshape(_N, _NS)

    y0lo, y0hi, y1lo, y1hi, d0, d1, d0bc, d1bc = _prep_call(hp0, hp1, x)

    w0lo, w0hi, w1lo, w1hi = _prop_call(y0lo, y0hi, y1lo, y1hi,
                                        e0, e1, d0bc, d1bc)

    a2d = a.reshape(1, 2).astype(jnp.float32)
    return _final_call(w0lo, w0hi, w1lo, w1hi, d0, d1,
                       W_enc0, b_enc0.reshape(1, _D),
                       W_enc1, b_enc1.reshape(1, _D),
                       a2d, W_dec, b_dec.reshape(1, _D))


# speedup vs baseline: 1.4263x; 1.0529x over previous
import functools

import jax
import jax.numpy as jnp
from jax import lax
from jax.experimental import pallas as pl
from jax.experimental.pallas import tpu as pltpu
from jax.experimental.pallas import tpu_sc as plsc

_N = 10000
_D = 128
_E = 320000

_NC = 2
_NS = 16
_HALF = _D // 2

_CHUNK = 512
_IDX_ROWS = _CHUNK // 128
_ER = 2 * _E // 256
_RPT = _ER // _NS
_STEPS = _RPT // _IDX_ROWS
_XTRA = (_ER - _RPT * _NS) // _IDX_ROWS
_NPT = _N // _NS

_HIST_ROWS = _N // 16

_BN = 2000


def _deg_body(e0_2d, e1_2d, h0_out, h1_out, colbuf, hist):
    c = lax.axis_index("c")
    s = lax.axis_index("s")
    ones = jnp.full((16,), 1.0, dtype=jnp.float32)

    def run(e2d, hout):
        def zero_row(i, carry):
            hist[i] = jnp.zeros((16,), dtype=jnp.float32)
            return carry
        lax.fori_loop(0, _HIST_ROWS, zero_row, 0)

        def step(t, carry):
            base = _ER + _RPT * s + t * _IDX_ROWS
            pltpu.sync_copy(e2d.at[pl.ds(base, _IDX_ROWS)], colbuf)
            for j in range(_IDX_ROWS):
                for k in range(8):
                    v = colbuf[j, pl.ds(k * 16, 16)]
                    r = lax.shift_right_logical(v, 4)
                    cc = lax.bitwise_and(v, 15)
                    plsc.addupdate_scatter(hist, [r, cc], ones)
            return carry
        lax.fori_loop(0, _STEPS, step, 0)

        @pl.when(s == _NS - 1)
        def _():
            lax.fori_loop(_STEPS, _STEPS + _XTRA, step, 0)

        pltpu.sync_copy(hist.at[pl.ds(0, _HIST_ROWS)], hout.at[s])

    @pl.when(c == 0)
    def _():
        run(e0_2d, h0_out)

    @pl.when(c == 1)
    def _():
        run(e1_2d, h1_out)


def _deg_call(e0_2d, e1_2d):
    mesh = plsc.VectorSubcoreMesh(core_axis_name="c", subcore_axis_name="s", num_cores=_NC, num_subcores=_NS)
    f = pl.kernel(
        _deg_body,
        out_type=[
            jax.ShapeDtypeStruct((_NS, _HIST_ROWS, 16), jnp.float32),
            jax.ShapeDtypeStruct((_NS, _HIST_ROWS, 16), jnp.float32),
        ],
        mesh=mesh,
        scratch_types=[
            pltpu.VMEM((_IDX_ROWS, 128), jnp.int32),
            pltpu.VMEM((_HIST_ROWS, 16), jnp.float32),
        ],
        compiler_params=pltpu.CompilerParams(needs_layout_passes=False),
    )
    return f(e0_2d, e1_2d)


def _prop_body(y0, y1, e0_2d, e1_2d,
               d0bc, d1bc,
               o0, o1,
               cbuf0, cbuf1, rbuf0, rbuf1, rowsbuf, dbuf, ysp, acc,
               gsem, ssem, isem0, isem1):
    c = lax.axis_index("c")
    s = lax.axis_index("s")
    col = c * _HALF

    def one_pass(e2d):
        def idx_issue(t, cb, rb, sem):
            base = jnp.minimum(_RPT * s + t * _IDX_ROWS, _ER - _IDX_ROWS)
            pltpu.async_copy(e2d.at[pl.ds(_ER + base, _IDX_ROWS)], cb, sem)
            pltpu.async_copy(e2d.at[pl.ds(base, _IDX_ROWS)], rb, sem)

        def idx_wait(cb, rb, sem):
            pltpu.make_async_copy(e2d.at[pl.ds(0, _IDX_ROWS)], cb, sem).wait()
            pltpu.make_async_copy(e2d.at[pl.ds(0, _IDX_ROWS)], rb, sem).wait()

        def chunk(cb, rb):
            gh = [
                pltpu.async_copy(ysp.at[cb.at[j]],
                                 rowsbuf.at[pl.ds(j * 128, 128)], gsem)
                for j in range(_IDX_ROWS)
            ]
            for h in gh:
                h.wait()
            sh = [
                pltpu.async_copy(rowsbuf.at[pl.ds(j * 128, 128)],
                                 acc.at[rb.at[j]], ssem, add=True)
                for j in range(_IDX_ROWS)
            ]
            for h in sh:
                h.wait()

        idx_issue(0, cbuf0, rbuf0, isem0)

        def group(g, carry):
            t = 2 * g
            idx_issue(t + 1, cbuf1, rbuf1, isem1)
            idx_wait(cbuf0, rbuf0, isem0)
            chunk(cbuf0, rbuf0)
            idx_issue(t + 2, cbuf0, rbuf0, isem0)
            idx_wait(cbuf1, rbuf1, isem1)
            chunk(cbuf1, rbuf1)
            return carry

        lax.fori_loop(0, (_STEPS - 1) // 2, group, 0)
        idx_wait(cbuf0, rbuf0, isem0)
        chunk(cbuf0, rbuf0)

        @pl.when(s == _NS - 1)
        def _():
            idx_issue(_STEPS, cbuf1, rbuf1, isem1)
            idx_wait(cbuf1, rbuf1, isem1)
            chunk(cbuf1, rbuf1)

    def run_rel(y_hbm, dbc_hbm, out_hbm, e2d):
        pltpu.sync_copy(y_hbm.at[pl.ds(s * _NPT, _NPT), pl.ds(col, _HALF)],
                        ysp.at[pl.ds(s * _NPT, _NPT)])
        pltpu.sync_copy(y_hbm.at[pl.ds(s * _NPT, _NPT), pl.ds(col, _HALF)],
                        acc.at[pl.ds(s * _NPT, _NPT)])
        pltpu.sync_copy(dbc_hbm.at[pl.ds(s * _NPT, _NPT)], dbuf)
        plsc.subcore_barrier()

        one_pass(e2d)
        plsc.subcore_barrier()

        for c0, ln in ((0, _CHUNK), (_CHUNK, _NPT - _CHUNK)):
            pltpu.sync_copy(acc.at[pl.ds(s * _NPT + c0, ln)],
                            rowsbuf.at[pl.ds(0, ln)])

            def scale_row(i, carry):
                m = dbuf[c0 + i]
                for k in range(_HALF // 16):
                    sl = pl.ds(k * 16, 16)
                    rowsbuf[i, sl] = rowsbuf[i, sl] * m
                return carry

            lax.fori_loop(0, ln, scale_row, 0)
            pltpu.sync_copy(rowsbuf.at[pl.ds(0, ln)],
                            ysp.at[pl.ds(s * _NPT + c0, ln)])
            pltpu.sync_copy(rowsbuf.at[pl.ds(0, ln)],
                            acc.at[pl.ds(s * _NPT + c0, ln)])
        plsc.subcore_barrier()

        one_pass(e2d)
        plsc.subcore_barrier()
        pltpu.sync_copy(acc.at[pl.ds(s * _NPT, _NPT)],
                        out_hbm.at[pl.ds(s * _NPT, _NPT), pl.ds(col, _HALF)])

    run_rel(y0, d0bc, o0, e0_2d)
    run_rel(y1, d1bc, o1, e1_2d)


def _prop_call(y0, y1, e0_2d, e1_2d, d0bc, d1bc):
    mesh = plsc.VectorSubcoreMesh(core_axis_name="c", subcore_axis_name="s", num_cores=_NC, num_subcores=_NS)
    full = jax.ShapeDtypeStruct((_N, _D), jnp.float32)
    f = pl.kernel(
        _prop_body,
        out_type=[full, full],
        mesh=mesh,
        scratch_types=[
            pltpu.VMEM((_IDX_ROWS, 128), jnp.int32),
            pltpu.VMEM((_IDX_ROWS, 128), jnp.int32),
            pltpu.VMEM((_IDX_ROWS, 128), jnp.int32),
            pltpu.VMEM((_IDX_ROWS, 128), jnp.int32),
            pltpu.VMEM((_CHUNK, _HALF), jnp.float32),
            pltpu.VMEM((_NPT, 16), jnp.float32),
            pltpu.VMEM_SHARED((_N, _HALF), jnp.float32),
            pltpu.VMEM_SHARED((_N, _HALF), jnp.float32),
            pltpu.SemaphoreType.DMA,
            pltpu.SemaphoreType.DMA,
            pltpu.SemaphoreType.DMA,
            pltpu.SemaphoreType.DMA,
        ],
        compiler_params=pltpu.CompilerParams(use_tc_tiling_on_sc=False),
    )
    return f(y0, y1, e0_2d, e1_2d, d0bc, d1bc)


def _prep_body(h0_ref, h1_ref, x_ref,
               y0_ref, y1_ref, d0_ref, d1_ref, d0bc_ref, d1bc_ref):
    deg0 = 1.0 + jnp.sum(h0_ref[...], axis=1, keepdims=True)
    deg1 = 1.0 + jnp.sum(h1_ref[...], axis=1, keepdims=True)
    dinv0 = jnp.where(deg0 > 0, lax.rsqrt(deg0), 0.0)
    dinv1 = jnp.where(deg1 > 0, lax.rsqrt(deg1), 0.0)
    x = x_ref[...]
    y0_ref[...] = x * dinv0
    y1_ref[...] = x * dinv1
    d0_ref[...] = dinv0
    d1_ref[...] = dinv1
    d0bc_ref[...] = jnp.broadcast_to(dinv0 * dinv0, (_BN, 16))
    d1bc_ref[...] = jnp.broadcast_to(dinv1 * dinv1, (_BN, 16))


def _prep_call(hp0, hp1, x):
    grid = (_N // _BN,)
    full_spec = pl.BlockSpec((_BN, _D), lambda i: (i, 0))
    f = pl.pallas_call(
        _prep_body,
        grid=grid,
        in_specs=[
            pl.BlockSpec((_BN, 16), lambda i: (i, 0)),
            pl.BlockSpec((_BN, 16), lambda i: (i, 0)),
            full_spec,
        ],
        out_specs=[
            full_spec, full_spec,
            pl.BlockSpec((_BN, 1), lambda i: (i, 0)),
            pl.BlockSpec((_BN, 1), lambda i: (i, 0)),
            pl.BlockSpec((_BN, 16), lambda i: (i, 0)),
            pl.BlockSpec((_BN, 16), lambda i: (i, 0)),
        ],
        out_shape=[
            jax.ShapeDtypeStruct((_N, _D), jnp.float32),
            jax.ShapeDtypeStruct((_N, _D), jnp.float32),
            jax.ShapeDtypeStruct((_N, 1), jnp.float32),
            jax.ShapeDtypeStruct((_N, 1), jnp.float32),
            jax.ShapeDtypeStruct((_N, 16), jnp.float32),
            jax.ShapeDtypeStruct((_N, 16), jnp.float32),
        ],
    )
    return f(hp0, hp1, x)


def _final_body(w0_ref, w1_ref, d0_ref, d1_ref,
                W0_ref, b0_ref, W1_ref, b1_ref, a_ref, Wd_ref, bd_ref,
                out_ref):
    h0 = w0_ref[...] * d0_ref[...]
    h1 = w1_ref[...] * d1_ref[...]
    e0 = jnp.dot(h0, W0_ref[...], preferred_element_type=jnp.float32,
                 precision=lax.Precision.HIGHEST) + b0_ref[...]
    e1 = jnp.dot(h1, W1_ref[...], preferred_element_type=jnp.float32,
                 precision=lax.Precision.HIGHEST) + b1_ref[...]
    a0 = a_ref[0, 0]
    a1 = a_ref[0, 1]
    m = jnp.maximum(a0, a1)
    x0 = jnp.exp(a0 - m)
    x1 = jnp.exp(a1 - m)
    ws0 = x0 / (x0 + x1)
    ws1 = x1 / (x0 + x1)
    fused = ws0 * e0 + ws1 * e1
    out_ref[...] = jnp.dot(fused, Wd_ref[...], preferred_element_type=jnp.float32,
                           precision=lax.Precision.HIGHEST) + bd_ref[...]


def _final_call(w0, w1, d0, d1, W0, b0, W1, b1, a2d, Wd, bd):
    grid = (_N // _BN,)
    full_spec = pl.BlockSpec((_BN, _D), lambda i: (i, 0))
    dspec = pl.BlockSpec((_BN, 1), lambda i: (i, 0))
    wspec = pl.BlockSpec((_D, _D), lambda i: (0, 0))
    bspec = pl.BlockSpec((1, _D), lambda i: (0, 0))
    f = pl.pallas_call(
        _final_body,
        grid=grid,
        in_specs=[
            full_spec, full_spec, dspec, dspec,
            wspec, bspec, wspec, bspec,
            pl.BlockSpec((1, 2), lambda i: (0, 0)),
            wspec, bspec,
        ],
        out_specs=pl.BlockSpec((_BN, _D), lambda i: (i, 0)),
        out_shape=jax.ShapeDtypeStruct((_N, _D), jnp.float32),
    )
    return f(w0, w1, d0, d1, W0, b0, W1, b1, a2d, Wd, bd)


@jax.jit
def kernel(x, edge_index_0, edge_index_1, W_enc0, b_enc0, W_enc1, b_enc1,
           a, W_dec, b_dec):
    e0 = edge_index_0.astype(jnp.int32).reshape(2 * _ER, 128)
    e1 = edge_index_1.astype(jnp.int32).reshape(2 * _ER, 128)

    hp0, hp1 = _deg_call(e0, e1)
    hp0 = jnp.transpose(hp0, (1, 2, 0)).reshape(_N, _NS)
    hp1 = jnp.transpose(hp1, (1, 2, 0)).reshape(_N, _NS)

    y0, y1, d0, d1, d0bc, d1bc = _prep_call(hp0, hp1, x)

    w0, w1 = _prop_call(y0, y1, e0, e1, d0bc, d1bc)

    a2d = a.reshape(1, 2).astype(jnp.float32)
    return _final_call(w0, w1, d0, d1,
                       W_enc0, b_enc0.reshape(1, _D),
                       W_enc1, b_enc1.reshape(1, _D),
                       a2d, W_dec, b_dec.reshape(1, _D))
